# baseline (speedup 1.0000x reference)
import jax
import jax.numpy as jnp
from jax.experimental import pallas as pl

N = 10000
G = 128


def _relu_body(x_ref, o_ref):
    o_ref[...] = jnp.maximum(x_ref[...], 0.0)


def _prelu(x):
    n, f = x.shape
    blk = 1000
    return pl.pallas_call(
        _relu_body,
        out_shape=jax.ShapeDtypeStruct((n, f), x.dtype),
        grid=(n // blk,),
        in_specs=[pl.BlockSpec((blk, f), lambda i: (i, 0))],
        out_specs=pl.BlockSpec((blk, f), lambda i: (i, 0)),
    )(x)


def _gcn_conv(x, W, b, row, col, ew):
    x = x @ W
    loop = jnp.arange(N, dtype=row.dtype)
    r = jnp.concatenate([row, loop])
    c = jnp.concatenate([col, loop])
    w = jnp.concatenate([ew, jnp.ones((N,), dtype=ew.dtype)])
    deg = jax.ops.segment_sum(w, c, num_segments=N)
    dis = jnp.where(deg > 0, jax.lax.rsqrt(jnp.maximum(deg, 1e-12)), 0.0)
    norm = dis[r] * w * dis[c]
    msg = x[r] * norm[:, None]
    out = jax.ops.segment_sum(msg, c, num_segments=N)
    return out + b


def kernel(x, edge_index, edge_weight, batch, W1, b1, W2, b2, W3, b3, W4, b4, W5, b5, Wl1, bl1, Wl2, bl2):
    ew = edge_weight / jnp.maximum(jnp.linalg.norm(edge_weight), 1e-12)
    row, col = edge_index[0], edge_index[1]
    h = _prelu(_gcn_conv(x, W1, b1, row, col, ew))
    h = _prelu(_gcn_conv(h, W2, b2, row, col, ew))
    h = _prelu(_gcn_conv(h, W3, b3, row, col, ew))
    h = _prelu(_gcn_conv(h, W4, b4, row, col, ew))
    h = _prelu(_gcn_conv(h, W5, b5, row, col, ew))
    pooled = jax.ops.segment_sum(h, batch, num_segments=G)
    h = jax.nn.relu(pooled @ Wl1 + bl1)
    h = h @ Wl2 + bl2
    return jax.nn.log_softmax(h, axis=-1)


# speedup vs baseline: 5.2670x; 5.2670x over previous
import functools

import jax
import jax.numpy as jnp
from jax import lax
from jax.experimental import pallas as pl
from jax.experimental.pallas import tpu as pltpu
from jax.experimental.pallas import tpu_sc as plsc

N = 10000
E = 320000
F = 128
C = 32
G = 128

TILES = 32
K = 128
PER = E // TILES
CH = (PER + K - 1) // K
PERP = CH * K
NPAD = 10240
RPT = NPAD // 16
ZR = 128
LANES = 16


def _sc_agg_body(xwd_hbm, rows_hbm, cols_hbm, ews_hbm, out_hbm,
                 ridx_v, cidx_v, ew_v, rows_v, zero_v, acc_sh, sem):
    c = lax.axis_index("c")
    s = lax.axis_index("s")
    base = s * RPT

    def zrow(i, _):
        for j in range(8):
            zero_v[i, pl.ds(j * LANES, LANES)] = jnp.zeros((LANES,), jnp.float32)
        return 0
    lax.fori_loop(0, ZR, zrow, 0)
    for i in range(5):
        pltpu.sync_copy(zero_v, acc_sh.at[pl.ds(base + i * ZR, ZR)])
    plsc.subcore_barrier()

    w = c * 16 + s

    def sgrp(g, _):
        k16 = g * LANES
        ev = ew_v[pl.ds(k16, LANES)]
        for j in range(LANES):
            e = ev[j]
            for jc in range(8):
                sl = pl.ds(jc * LANES, LANES)
                rows_v[k16 + j, sl] = rows_v[k16 + j, sl] * e
        return 0

    def chunk(ci, _):
        pltpu.sync_copy(rows_hbm.at[w, ci], ridx_v)
        pltpu.sync_copy(cols_hbm.at[w, ci], cidx_v)
        pltpu.sync_copy(ews_hbm.at[w, ci], ew_v)
        pltpu.async_copy(xwd_hbm.at[ridx_v], rows_v, sem).wait()
        lax.fori_loop(0, K // LANES, sgrp, 0)
        pltpu.sync_copy(rows_v, acc_sh.at[cidx_v], add=True)
        return 0
    lax.fori_loop(0, CH, chunk, 0)

    plsc.subcore_barrier()
    pltpu.sync_copy(acc_sh.at[pl.ds(base, RPT)],
                    out_hbm.at[c, pl.ds(base, RPT)])


def _sc_agg(xwd, rows3, cols3, ews3):
    mesh = plsc.VectorSubcoreMesh(core_axis_name="c", subcore_axis_name="s")
    fn = functools.partial(
        pl.kernel, mesh=mesh,
        out_type=jax.ShapeDtypeStruct((2, NPAD, F), jnp.float32),
        scratch_types=[
            pltpu.VMEM((K,), jnp.int32),
            pltpu.VMEM((K,), jnp.int32),
            pltpu.VMEM((K,), jnp.float32),
            pltpu.VMEM((K, F), jnp.float32),
            pltpu.VMEM((ZR, F), jnp.float32),
            pltpu.VMEM_SHARED((NPAD, F), jnp.float32),
            pltpu.SemaphoreType.DMA,
        ],
    )(_sc_agg_body)
    return fn(xwd, rows3, cols3, ews3)


def _sumsq_body(x_ref, o_ref):
    blk = x_ref[...]
    o_ref[...] = jnp.sum(blk * blk).reshape(1, 1)


def _sumsq(ew2d):
    n = ew2d.shape[0]
    return pl.pallas_call(
        _sumsq_body,
        out_shape=jax.ShapeDtypeStruct((1, 1), jnp.float32),
        grid=(1,),
        in_specs=[pl.BlockSpec((n, 128), lambda i: (0, 0))],
        out_specs=pl.BlockSpec((1, 1), lambda i: (0, 0)),
    )(ew2d)


def _prep_body(degp_ref, s2_ref, o_ref):
    s = lax.rsqrt(jnp.maximum(s2_ref[...][0, 0], 1e-24))
    deg = s * (degp_ref[0] + degp_ref[1]) + 1.0
    o_ref[...] = lax.rsqrt(deg)


def _prep(degp, s2):
    blk = 1000
    return pl.pallas_call(
        _prep_body,
        out_shape=jax.ShapeDtypeStruct((N, F), jnp.float32),
        grid=(N // blk,),
        in_specs=[
            pl.BlockSpec((2, blk, F), lambda i: (0, i, 0)),
            pl.BlockSpec((1, 1), lambda i: (0, 0)),
        ],
        out_specs=pl.BlockSpec((blk, F), lambda i: (i, 0)),
    )(degp, s2)


def _mm1_body(x_ref, w_ref, dis_ref, o_ref):
    xw = jax.lax.dot_general(x_ref[...], w_ref[...], (((1,), (0,)), ((), ())),
                             precision=lax.Precision.HIGHEST,
                             preferred_element_type=jnp.float32)
    o_ref[...] = dis_ref[...] * xw


def _mm1(x, W, dis):
    blk = 1000
    fin = x.shape[1]
    return pl.pallas_call(
        _mm1_body,
        out_shape=jax.ShapeDtypeStruct((N, F), jnp.float32),
        grid=(N // blk,),
        in_specs=[
            pl.BlockSpec((blk, fin), lambda i: (i, 0)),
            pl.BlockSpec((fin, F), lambda i: (0, 0)),
            pl.BlockSpec((blk, F), lambda i: (i, 0)),
        ],
        out_specs=pl.BlockSpec((blk, F), lambda i: (i, 0)),
    )(x, W, dis)


def _layer_body(agg_ref, xwd_ref, dis_ref, s2_ref, b_ref, wn_ref, o_ref):
    s = lax.rsqrt(jnp.maximum(s2_ref[...][0, 0], 1e-24))
    dis = dis_ref[...]
    h = dis * (s * (agg_ref[0] + agg_ref[1]) + xwd_ref[...]) + b_ref[...]
    h = jnp.maximum(h, 0.0)
    hw = jax.lax.dot_general(h, wn_ref[...], (((1,), (0,)), ((), ())),
                             precision=lax.Precision.HIGHEST,
                             preferred_element_type=jnp.float32)
    o_ref[...] = dis * hw


def _layer(agg, xwd, dis, s2, b, Wn):
    blk = 1000
    return pl.pallas_call(
        _layer_body,
        out_shape=jax.ShapeDtypeStruct((N, F), jnp.float32),
        grid=(N // blk,),
        in_specs=[
            pl.BlockSpec((2, blk, F), lambda i: (0, i, 0)),
            pl.BlockSpec((blk, F), lambda i: (i, 0)),
            pl.BlockSpec((blk, F), lambda i: (i, 0)),
            pl.BlockSpec((1, 1), lambda i: (0, 0)),
            pl.BlockSpec((1, F), lambda i: (0, 0)),
            pl.BlockSpec((F, F), lambda i: (0, 0)),
        ],
        out_specs=pl.BlockSpec((blk, F), lambda i: (i, 0)),
    )(agg, xwd, dis, s2, b, Wn)


def _pool_body(agg_ref, xwd_ref, dis_ref, s2_ref, b_ref, batch_ref, o_ref):
    @pl.when(pl.program_id(0) == 0)
    def _():
        o_ref[...] = jnp.zeros_like(o_ref)
    s = lax.rsqrt(jnp.maximum(s2_ref[...][0, 0], 1e-24))
    dis = dis_ref[...]
    h = dis * (s * (agg_ref[0] + agg_ref[1]) + xwd_ref[...]) + b_ref[...]
    h = jnp.maximum(h, 0.0)
    gids = jax.lax.broadcasted_iota(jnp.int32, (1, G), 1)
    onehot = (batch_ref[...] == gids).astype(jnp.float32)
    o_ref[...] += jax.lax.dot_general(
        onehot, h, (((0,), (0,)), ((), ())),
        precision=lax.Precision.HIGHEST,
        preferred_element_type=jnp.float32)


def _pool(agg, xwd, dis, s2, b, batch2d):
    blk = 1000
    return pl.pallas_call(
        _pool_body,
        out_shape=jax.ShapeDtypeStruct((G, F), jnp.float32),
        grid=(N // blk,),
        in_specs=[
            pl.BlockSpec((2, blk, F), lambda i: (0, i, 0)),
            pl.BlockSpec((blk, F), lambda i: (i, 0)),
            pl.BlockSpec((blk, F), lambda i: (i, 0)),
            pl.BlockSpec((1, 1), lambda i: (0, 0)),
            pl.BlockSpec((1, F), lambda i: (0, 0)),
            pl.BlockSpec((blk, 1), lambda i: (i, 0)),
        ],
        out_specs=pl.BlockSpec((G, F), lambda i: (0, 0)),
    )(agg, xwd, dis, s2, b, batch2d)


def _head_body(p_ref, w1_ref, b1_ref, w2_ref, b2_ref, o_ref):
    h1 = jax.lax.dot_general(p_ref[...], w1_ref[...], (((1,), (0,)), ((), ())),
                             precision=lax.Precision.HIGHEST,
                             preferred_element_type=jnp.float32)
    h1 = jnp.maximum(h1 + b1_ref[...], 0.0)
    t = jax.lax.dot_general(h1, w2_ref[...], (((1,), (0,)), ((), ())),
                            precision=lax.Precision.HIGHEST,
                            preferred_element_type=jnp.float32) + b2_ref[...]
    m = jnp.max(t, axis=-1, keepdims=True)
    lse = jnp.log(jnp.sum(jnp.exp(t - m), axis=-1, keepdims=True)) + m
    o_ref[...] = t - lse


def _head(pooled, Wl1, bl1, Wl2p, bl2p):
    return pl.pallas_call(
        _head_body,
        out_shape=jax.ShapeDtypeStruct((G, F), jnp.float32),
        in_specs=[pl.BlockSpec(a.shape, lambda: tuple(0 for _ in a.shape))
                  for a in (pooled, Wl1, bl1, Wl2p, bl2p)],
        out_specs=pl.BlockSpec((G, F), lambda: (0, 0)),
    )(pooled, Wl1, bl1, Wl2p, bl2p)


def kernel(x, edge_index, edge_weight, batch,
           W1, b1, W2, b2, W3, b3, W4, b4, W5, b5, Wl1, bl1, Wl2, bl2):
    pad = TILES * PERP - E
    row = jnp.concatenate([edge_index[0], jnp.zeros((pad,), jnp.int32)])
    col = jnp.concatenate([edge_index[1], jnp.zeros((pad,), jnp.int32)])
    ewp = jnp.concatenate([edge_weight, jnp.zeros((pad,), jnp.float32)])
    rows3 = row.reshape(TILES, CH, K)
    cols3 = col.reshape(TILES, CH, K)
    ews3 = ewp.reshape(TILES, CH, K)

    s2 = _sumsq(edge_weight.reshape(2500, 128))
    degp = _sc_agg(jnp.ones((N, F), jnp.float32), rows3, cols3, ews3)
    dis = _prep(degp, s2)

    batch2d = batch.reshape(N, 1)
    bs = [b1, b2, b3, b4, b5]
    Ws = [W2, W3, W4, W5]

    xwd = _mm1(x, W1, dis)
    for l in range(4):
        agg = _sc_agg(xwd, rows3, cols3, ews3)
        xwd = _layer(agg, xwd, dis, s2, bs[l].reshape(1, F), Ws[l])
    agg = _sc_agg(xwd, rows3, cols3, ews3)
    pooled = _pool(agg, xwd, dis, s2, bs[4].reshape(1, F), batch2d)

    Wl2p = jnp.zeros((F, F), jnp.float32).at[:, :C].set(Wl2)
    bl2p = jnp.full((1, F), -1e30, jnp.float32).at[0, :C].set(bl2)
    out = _head(pooled, Wl1, bl1.reshape(1, F), Wl2p, bl2p)
    return out[:, :C]
